# baseline (speedup 1.0000x reference)
import functools

import jax
import jax.numpy as jnp
from jax import lax
from jax.experimental import pallas as pl
from jax.experimental.pallas import tpu as pltpu

N = 5000
E = 20000
D_IN = 128
H = 64
G = 4 * H
NE = E + N
TOTPAD = 165888
CPAD = 5120
CW = 512
NCHUNK = 10
ETS = ("ss", "tt", "st", "ts")


def _sigmoid(x):
    return 1.0 / (1.0 + jnp.exp(-x))


def _enc_kernel(xs_ref, xt_ref, ws1, bs1, ws2, bs2, wt1, bt1, wt2, bt2,
                os_ref, ot_ref):
    hs = jnp.maximum(
        jnp.dot(xs_ref[...], ws1[...], preferred_element_type=jnp.float32)
        + bs1[...], 0.0)
    os_ref[...] = jnp.dot(hs, ws2[...], preferred_element_type=jnp.float32) + bs2[...]
    ht = jnp.maximum(
        jnp.dot(xt_ref[...], wt1[...], preferred_element_type=jnp.float32)
        + bt1[...], 0.0)
    ot_ref[...] = jnp.dot(ht, wt2[...], preferred_element_type=jnp.float32) + bt2[...]


def _encode(x_source, x_target, ps, pt):
    return pl.pallas_call(
        _enc_kernel,
        out_shape=(jax.ShapeDtypeStruct((N, H), jnp.float32),
                   jax.ShapeDtypeStruct((N, H), jnp.float32)),
    )(x_source, x_target,
      ps["W1"], ps["b1"].reshape(1, H), ps["W2"], ps["b2"].reshape(1, H),
      pt["W1"], pt["b1"].reshape(1, H), pt["W2"], pt["b2"].reshape(1, H))


def _p_kernel(f_ref, w_ref, b_ref, p_ref):
    p_ref[...] = (
        jnp.dot(f_ref[...], w_ref[...], preferred_element_type=jnp.float32)
        + b_ref[...])


def _compute_p(fs, ft, wih, bb):
    f2 = jnp.stack([fs, ft])
    return pl.pallas_call(
        _p_kernel,
        grid=(4,),
        in_specs=[
            pl.BlockSpec((None, N, H), lambda i: (i % 2, 0, 0)),
            pl.BlockSpec((None, H, G), lambda i: (i, 0, 0)),
            pl.BlockSpec((None, 1, G), lambda i: (i, 0, 0)),
        ],
        out_specs=pl.BlockSpec((None, N, G), lambda i: (i, 0, 0)),
        out_shape=jax.ShapeDtypeStruct((4, N, G), jnp.float32),
    )(f2, wih, bb)


def _lstm_kernel(t_ref, r_hbm, cnt_ref, whh_ref, bb_ref, h_ref, x_scr, sems):
    big_t = t_ref[0, 0]
    whh = whh_ref[...]
    bb = bb_ref[...]
    cnts = cnt_ref[...]
    row = lax.broadcasted_iota(jnp.int32, (N, 1), 0)

    def fetch(off, n_t):
        off = pl.multiple_of(off, 8)
        for k in range(NCHUNK):
            @pl.when(k * CW < n_t)
            def _():
                pltpu.make_async_copy(
                    r_hbm.at[pl.ds(off + k * CW, CW)],
                    x_scr.at[pl.ds(k * CW, CW)],
                    sems.at[k],
                ).start()

    def drain(off, n_t):
        off = pl.multiple_of(off, 8)
        for k in range(NCHUNK):
            @pl.when(k * CW < n_t)
            def _():
                pltpu.make_async_copy(
                    r_hbm.at[pl.ds(off + k * CW, CW)],
                    x_scr.at[pl.ds(k * CW, CW)],
                    sems.at[k],
                ).wait()

    def step(t, carry):
        h, c, off = carry
        n_t = jnp.sum((cnts > t).astype(jnp.int32))
        fetch(off, n_t)
        drain(off, n_t)
        x = x_scr[0:N, :]
        g = (jnp.dot(h, whh, preferred_element_type=jnp.float32)
             + jnp.where(row < n_t, x, bb))
        i = _sigmoid(g[:, 0:H])
        f = _sigmoid(g[:, H:2 * H])
        gg = jnp.tanh(g[:, 2 * H:3 * H])
        o = _sigmoid(g[:, 3 * H:4 * H])
        c = f * c + i * gg
        h = o * jnp.tanh(c)
        npad = ((n_t + 7) // 8) * 8
        return (h, c, off + npad)

    z = jnp.zeros((N, H), jnp.float32)
    h, _, _ = lax.fori_loop(0, big_t, step, (z, z, jnp.int32(0)))
    h_ref[...] = h


def _lstm(t_arr, r, cnt2d, whh, bb):
    return pl.pallas_call(
        _lstm_kernel,
        in_specs=[
            pl.BlockSpec(memory_space=pltpu.SMEM),
            pl.BlockSpec(memory_space=pl.ANY),
            pl.BlockSpec(memory_space=pltpu.VMEM),
            pl.BlockSpec(memory_space=pltpu.VMEM),
            pl.BlockSpec(memory_space=pltpu.VMEM),
        ],
        out_shape=jax.ShapeDtypeStruct((N, H), jnp.float32),
        scratch_shapes=[
            pltpu.VMEM((NCHUNK * CW, G), jnp.float32),
            pltpu.SemaphoreType.DMA((NCHUNK,)),
        ],
    )(t_arr, r, cnt2d, whh, bb)


def _combine_kernel(hss, htt, hst, hts, fs, ft, wl, bl, wr, ps_ref, pt_ref,
                    os_ref, ot_ref):
    def lin(h, xd, k):
        return (jnp.dot(h[...], wl[k], preferred_element_type=jnp.float32)
                + bl[k]
                + jnp.dot(xd, wr[k], preferred_element_type=jnp.float32))

    xs = fs[...]
    xt = ft[...]
    o_ss = lin(hss, xs, 0)
    o_tt = lin(htt, xt, 1)
    o_st = lin(hst, xt, 2)
    o_ts = lin(hts, xs, 3)
    os_ref[...] = jnp.maximum((o_ss + o_ts) * 0.5 + ps_ref[...], 0.0)
    ot_ref[...] = jnp.maximum((o_tt + o_st) * 0.5 + pt_ref[...], 0.0)


def _combine(hs, fs, ft, wl, bl, wr, prev_s, prev_t):
    return pl.pallas_call(
        _combine_kernel,
        out_shape=(jax.ShapeDtypeStruct((N, H), jnp.float32),
                   jax.ShapeDtypeStruct((N, H), jnp.float32)),
    )(hs[0], hs[1], hs[2], hs[3], fs, ft, wl, bl, wr, prev_s, prev_t)


def _prep(edge_index):
    ar = jnp.arange(N, dtype=jnp.int32)
    src = jnp.concatenate([edge_index[0].astype(jnp.int32), ar])
    dst = jnp.concatenate([edge_index[1].astype(jnp.int32), ar])
    order = jnp.argsort(dst, stable=True)
    dst_s = dst[order]
    src_s = src[order]
    counts = jnp.zeros((N,), jnp.int32).at[dst].add(1)
    starts = jnp.cumsum(counts) - counts
    big_t = jnp.max(counts)
    perm = jnp.argsort(-counts, stable=True)
    rank = jnp.zeros((N,), jnp.int32).at[perm].set(ar)
    t_seg = jnp.arange(NE, dtype=jnp.int32) - starts[dst_s]
    key = t_seg * 8192 + rank[dst_s]
    order2 = jnp.argsort(key)
    gidx = src_s[order2]
    t_sorted = t_seg[order2]
    n_of_t = jnp.zeros((NE,), jnp.int32).at[t_sorted].add(1)
    grp8 = ((n_of_t + 7) // 8) * 8
    off_pad = jnp.cumsum(grp8) - grp8
    off_raw = jnp.cumsum(n_of_t) - n_of_t
    pos = off_pad[t_sorted] + (jnp.arange(NE, dtype=jnp.int32) - off_raw[t_sorted])
    inv = jnp.zeros((TOTPAD,), jnp.int32).at[pos].set(jnp.arange(NE, dtype=jnp.int32))
    gidx_pad = gidx[inv]
    cnt2d = jnp.pad(counts[perm], (0, CPAD - N)).reshape(40, 128)
    return gidx_pad, cnt2d, rank, big_t.reshape(1, 1).astype(jnp.int32)


def _layer(prep, fs, ft, lp, prev_s, prev_t):
    wih = jnp.stack([lp[et]["Wih"] for et in ETS])
    bb = jnp.stack([(lp[et]["bih"] + lp[et]["bhh"]).reshape(1, G) for et in ETS])
    wl = jnp.stack([lp[et]["Wl"] for et in ETS])
    bl = jnp.stack([lp[et]["bl"].reshape(1, H) for et in ETS])
    wr = jnp.stack([lp[et]["Wr"] for et in ETS])
    p_all = _compute_p(fs, ft, wih, bb)
    hs = []
    for k in range(4):
        gidx_pad, cnt2d, rank, t_arr = prep[k]
        r = jnp.take(p_all[k], gidx_pad, axis=0)
        h_perm = _lstm(t_arr, r, cnt2d, lp[ETS[k]]["Whh"], bb[k])
        hs.append(jnp.take(h_perm, rank, axis=0))
    return _combine(hs, fs, ft, wl, bl, wr, prev_s, prev_t)


def kernel(x_source, x_target, edge_index_ss, edge_index_tt, edge_index_st,
           edge_index_ts, edge_attr_ss, edge_attr_tt, edge_attr_st,
           edge_attr_ts, params):
    prep = [_prep(ei) for ei in
            (edge_index_ss, edge_index_tt, edge_index_st, edge_index_ts)]
    fs, ft = _encode(x_source, x_target, params["src_enc"], params["tgt_enc"])
    zero = jnp.zeros((N, H), jnp.float32)
    s1, t1 = _layer(prep, fs, ft, params["conv1"], zero, zero)
    s2, t2 = _layer(prep, s1, t1, params["conv2"], s1, t1)
    s3, t3 = _layer(prep, s2, t2, params["conv3"], s2, t2)
    return s3, t3


# speedup vs baseline: 3.3003x; 3.3003x over previous
import functools

import jax
import jax.numpy as jnp
from jax import lax
from jax.experimental import pallas as pl
from jax.experimental.pallas import tpu as pltpu
from jax.experimental.pallas import tpu_sc as plsc

N = 5000
E = 20000
D_IN = 128
H = 64
G = 4 * H
NE = E + N
TOTPAD = 165888
CPAD = 5120
CW = 512
NCHUNK = 10
ETS = ("ss", "tt", "st", "ts")
SC_NW = 32
EPAD = 25600
PERW = 4 * EPAD // SC_NW
SCCH = 128
NSCCH = PERW // SCCH


def _sigmoid(x):
    return 1.0 / (1.0 + jnp.exp(-x))


def _enc_kernel(xs_ref, xt_ref, ws1, bs1, ws2, bs2, wt1, bt1, wt2, bt2,
                os_ref, ot_ref):
    hs = jnp.maximum(
        jnp.dot(xs_ref[...], ws1[...], preferred_element_type=jnp.float32)
        + bs1[...], 0.0)
    os_ref[...] = jnp.dot(hs, ws2[...], preferred_element_type=jnp.float32) + bs2[...]
    ht = jnp.maximum(
        jnp.dot(xt_ref[...], wt1[...], preferred_element_type=jnp.float32)
        + bt1[...], 0.0)
    ot_ref[...] = jnp.dot(ht, wt2[...], preferred_element_type=jnp.float32) + bt2[...]


def _encode(x_source, x_target, ps, pt):
    return pl.pallas_call(
        _enc_kernel,
        out_shape=(jax.ShapeDtypeStruct((N, H), jnp.float32),
                   jax.ShapeDtypeStruct((N, H), jnp.float32)),
    )(x_source, x_target,
      ps["W1"], ps["b1"].reshape(1, H), ps["W2"], ps["b2"].reshape(1, H),
      pt["W1"], pt["b1"].reshape(1, H), pt["W2"], pt["b2"].reshape(1, H))


def _p_kernel(f_ref, w_ref, b_ref, p_ref):
    p_ref[...] = (
        jnp.dot(f_ref[...], w_ref[...], preferred_element_type=jnp.float32)
        + b_ref[...])


def _compute_p(fs, ft, wih, bb):
    f2 = jnp.stack([fs, ft])
    return pl.pallas_call(
        _p_kernel,
        grid=(4,),
        in_specs=[
            pl.BlockSpec((None, N, H), lambda i: (i % 2, 0, 0)),
            pl.BlockSpec((None, H, G), lambda i: (i, 0, 0)),
            pl.BlockSpec((None, 1, G), lambda i: (i, 0, 0)),
        ],
        out_specs=pl.BlockSpec((None, N, G), lambda i: (i, 0, 0)),
        out_shape=jax.ShapeDtypeStruct((4, N, G), jnp.float32),
    )(f2, wih, bb)


def _lstm_kernel(t_ref, r_hbm, cnt_ref, whh_ref, bb_ref, h_ref, x_scr, sems,
                 *, base):
    big_t = t_ref[0, 0]
    whh = whh_ref[...]
    bb = bb_ref[...]
    cnts = cnt_ref[...]
    row = lax.broadcasted_iota(jnp.int32, (N, 1), 0)

    def fetch(off, n_t):
        off = pl.multiple_of(off, 8)
        for k in range(NCHUNK):
            @pl.when(k * CW < n_t)
            def _():
                pltpu.make_async_copy(
                    r_hbm.at[pl.ds(base + off + k * CW, CW)],
                    x_scr.at[pl.ds(k * CW, CW)],
                    sems.at[k],
                ).start()

    def drain(off, n_t):
        off = pl.multiple_of(off, 8)
        for k in range(NCHUNK):
            @pl.when(k * CW < n_t)
            def _():
                pltpu.make_async_copy(
                    r_hbm.at[pl.ds(base + off + k * CW, CW)],
                    x_scr.at[pl.ds(k * CW, CW)],
                    sems.at[k],
                ).wait()

    def step(t, carry):
        h, c, off = carry
        n_t = jnp.sum((cnts > t).astype(jnp.int32))
        fetch(off, n_t)
        drain(off, n_t)
        x = x_scr[0:N, :]
        g = (jnp.dot(h, whh, preferred_element_type=jnp.float32)
             + jnp.where(row < n_t, x, bb))
        i = _sigmoid(g[:, 0:H])
        f = _sigmoid(g[:, H:2 * H])
        gg = jnp.tanh(g[:, 2 * H:3 * H])
        o = _sigmoid(g[:, 3 * H:4 * H])
        c = f * c + i * gg
        h = o * jnp.tanh(c)
        npad = ((n_t + 7) // 8) * 8
        return (h, c, off + npad)

    z = jnp.zeros((N, H), jnp.float32)
    h, _, _ = lax.fori_loop(0, big_t, step, (z, z, jnp.int32(0)))
    h_ref[...] = h


def _lstm(t_arr, r, cnt2d, whh, bb, base):
    return pl.pallas_call(
        functools.partial(_lstm_kernel, base=base),
        in_specs=[
            pl.BlockSpec(memory_space=pltpu.SMEM),
            pl.BlockSpec(memory_space=pl.ANY),
            pl.BlockSpec(memory_space=pltpu.VMEM),
            pl.BlockSpec(memory_space=pltpu.VMEM),
            pl.BlockSpec(memory_space=pltpu.VMEM),
        ],
        out_shape=jax.ShapeDtypeStruct((N, H), jnp.float32),
        scratch_shapes=[
            pltpu.VMEM((NCHUNK * CW, G), jnp.float32),
            pltpu.SemaphoreType.DMA((NCHUNK,)),
        ],
    )(t_arr, r, cnt2d, whh, bb)


def _sc_gather_scatter(p_flat, gidx2, pos2):
    mesh = plsc.VectorSubcoreMesh(core_axis_name="c", subcore_axis_name="s")

    @functools.partial(
        pl.kernel, mesh=mesh,
        out_type=jax.ShapeDtypeStruct((4 * TOTPAD, G), jnp.float32),
        scratch_types=[
            pltpu.VMEM((NSCCH, SCCH), jnp.int32),
            pltpu.VMEM((NSCCH, SCCH), jnp.int32),
            pltpu.VMEM((SCCH, G), jnp.float32),
            pltpu.SemaphoreType.DMA,
        ],
    )
    def k(p_hbm, gidx_hbm, pos_hbm, out_hbm, idx_v, pos_v, rows_v, sem):
        wid = lax.axis_index("s") * 2 + lax.axis_index("c")
        pltpu.sync_copy(gidx_hbm.at[wid], idx_v)
        pltpu.sync_copy(pos_hbm.at[wid], pos_v)
        for i in range(NSCCH):
            pltpu.async_copy(p_hbm.at[idx_v.at[i]], rows_v, sem).wait()
            pltpu.async_copy(rows_v, out_hbm.at[pos_v.at[i]], sem).wait()

    return k(p_flat, gidx2, pos2)


def _combine_kernel(hss, htt, hst, hts, fs, ft, wl, bl, wr, ps_ref, pt_ref,
                    os_ref, ot_ref):
    def lin(h, xd, k):
        return (jnp.dot(h[...], wl[k], preferred_element_type=jnp.float32)
                + bl[k]
                + jnp.dot(xd, wr[k], preferred_element_type=jnp.float32))

    xs = fs[...]
    xt = ft[...]
    o_ss = lin(hss, xs, 0)
    o_tt = lin(htt, xt, 1)
    o_st = lin(hst, xt, 2)
    o_ts = lin(hts, xs, 3)
    os_ref[...] = jnp.maximum((o_ss + o_ts) * 0.5 + ps_ref[...], 0.0)
    ot_ref[...] = jnp.maximum((o_tt + o_st) * 0.5 + pt_ref[...], 0.0)


def _combine(hs, fs, ft, wl, bl, wr, prev_s, prev_t):
    return pl.pallas_call(
        _combine_kernel,
        out_shape=(jax.ShapeDtypeStruct((N, H), jnp.float32),
                   jax.ShapeDtypeStruct((N, H), jnp.float32)),
    )(hs[0], hs[1], hs[2], hs[3], fs, ft, wl, bl, wr, prev_s, prev_t)


def _prep(edge_index):
    ar = jnp.arange(N, dtype=jnp.int32)
    src = jnp.concatenate([edge_index[0].astype(jnp.int32), ar])
    dst = jnp.concatenate([edge_index[1].astype(jnp.int32), ar])
    order = jnp.argsort(dst, stable=True)
    dst_s = dst[order]
    src_s = src[order]
    counts = jnp.zeros((N,), jnp.int32).at[dst].add(1)
    starts = jnp.cumsum(counts) - counts
    big_t = jnp.max(counts)
    perm = jnp.argsort(-counts, stable=True)
    rank = jnp.zeros((N,), jnp.int32).at[perm].set(ar)
    t_seg = jnp.arange(NE, dtype=jnp.int32) - starts[dst_s]
    key = t_seg * 8192 + rank[dst_s]
    order2 = jnp.argsort(key)
    gidx = src_s[order2]
    t_sorted = t_seg[order2]
    n_of_t = jnp.zeros((NE,), jnp.int32).at[t_sorted].add(1)
    grp8 = ((n_of_t + 7) // 8) * 8
    off_pad = jnp.cumsum(grp8) - grp8
    off_raw = jnp.cumsum(n_of_t) - n_of_t
    pos = off_pad[t_sorted] + (jnp.arange(NE, dtype=jnp.int32) - off_raw[t_sorted])
    cnt2d = jnp.pad(counts[perm], (0, CPAD - N)).reshape(40, 128)
    return gidx, pos, cnt2d, rank, big_t.reshape(1, 1).astype(jnp.int32)


def _layer(prep, gidx2, pos2, fs, ft, lp, prev_s, prev_t):
    wih = jnp.stack([lp[et]["Wih"] for et in ETS])
    bb = jnp.stack([(lp[et]["bih"] + lp[et]["bhh"]).reshape(1, G) for et in ETS])
    wl = jnp.stack([lp[et]["Wl"] for et in ETS])
    bl = jnp.stack([lp[et]["bl"].reshape(1, H) for et in ETS])
    wr = jnp.stack([lp[et]["Wr"] for et in ETS])
    p_all = _compute_p(fs, ft, wih, bb)
    r_all = _sc_gather_scatter(p_all.reshape(4 * N, G), gidx2, pos2)
    hs = []
    for k in range(4):
        _, _, cnt2d, rank, t_arr = prep[k]
        h_perm = _lstm(t_arr, r_all, cnt2d, lp[ETS[k]]["Whh"], bb[k],
                       k * TOTPAD)
        hs.append(jnp.take(h_perm, rank, axis=0))
    return _combine(hs, fs, ft, wl, bl, wr, prev_s, prev_t)


def kernel(x_source, x_target, edge_index_ss, edge_index_tt, edge_index_st,
           edge_index_ts, edge_attr_ss, edge_attr_tt, edge_attr_st,
           edge_attr_ts, params):
    prep = [_prep(ei) for ei in
            (edge_index_ss, edge_index_tt, edge_index_st, edge_index_ts)]
    gparts, pparts = [], []
    for k in range(4):
        gidx, pos, _, _, _ = prep[k]
        gpad = jnp.full((EPAD - NE,), 0, jnp.int32)
        ppad = 165008 + jnp.arange(EPAD - NE, dtype=jnp.int32)
        gparts.append(jnp.concatenate([gidx + k * N, gpad + k * N]))
        pparts.append(jnp.concatenate([pos + k * TOTPAD, ppad + k * TOTPAD]))
    gidx2 = jnp.concatenate(gparts).reshape(SC_NW, NSCCH, SCCH)
    pos2 = jnp.concatenate(pparts).reshape(SC_NW, NSCCH, SCCH)
    fs, ft = _encode(x_source, x_target, params["src_enc"], params["tgt_enc"])
    zero = jnp.zeros((N, H), jnp.float32)
    s1, t1 = _layer(prep, gidx2, pos2, fs, ft, params["conv1"], zero, zero)
    s2, t2 = _layer(prep, gidx2, pos2, s1, t1, params["conv2"], s1, t1)
    s3, t3 = _layer(prep, gidx2, pos2, s2, t2, params["conv3"], s2, t2)
    return s3, t3


# speedup vs baseline: 8.9071x; 2.6988x over previous
import functools

import jax
import jax.numpy as jnp
from jax import lax
from jax.experimental import pallas as pl
from jax.experimental.pallas import tpu as pltpu
from jax.experimental.pallas import tpu_sc as plsc

N = 5000
E = 20000
D_IN = 128
H = 64
G = 4 * H
NE = E + N
TOTPAD = 165888
CPAD = 5120
CW = 512
NCHUNK = 10
ETS = ("ss", "tt", "st", "ts")
SC_NW = 32
EPAD = 25600
PERW = 4 * EPAD // SC_NW
SCCH = 128
NSCCH = PERW // SCCH
UET = 5120
NSCCHU = 4 * UET // SC_NW // SCCH


def _sigmoid(x):
    return 1.0 / (1.0 + jnp.exp(-x))


def _enc_kernel(xs_ref, xt_ref, ws1, bs1, ws2, bs2, wt1, bt1, wt2, bt2,
                os_ref, ot_ref):
    hs = jnp.maximum(
        jnp.dot(xs_ref[...], ws1[...], preferred_element_type=jnp.float32)
        + bs1[...], 0.0)
    os_ref[...] = jnp.dot(hs, ws2[...], preferred_element_type=jnp.float32) + bs2[...]
    ht = jnp.maximum(
        jnp.dot(xt_ref[...], wt1[...], preferred_element_type=jnp.float32)
        + bt1[...], 0.0)
    ot_ref[...] = jnp.dot(ht, wt2[...], preferred_element_type=jnp.float32) + bt2[...]


def _encode(x_source, x_target, ps, pt):
    return pl.pallas_call(
        _enc_kernel,
        out_shape=(jax.ShapeDtypeStruct((N, H), jnp.float32),
                   jax.ShapeDtypeStruct((N, H), jnp.float32)),
    )(x_source, x_target,
      ps["W1"], ps["b1"].reshape(1, H), ps["W2"], ps["b2"].reshape(1, H),
      pt["W1"], pt["b1"].reshape(1, H), pt["W2"], pt["b2"].reshape(1, H))


def _p_kernel(f_ref, w_ref, b_ref, p_ref):
    p_ref[...] = (
        jnp.dot(f_ref[...], w_ref[...], preferred_element_type=jnp.float32)
        + b_ref[...])


def _compute_p(fs, ft, wih, bb):
    f2 = jnp.stack([fs, ft])
    return pl.pallas_call(
        _p_kernel,
        grid=(4,),
        in_specs=[
            pl.BlockSpec((None, N, H), lambda i: (i % 2, 0, 0)),
            pl.BlockSpec((None, H, G), lambda i: (i, 0, 0)),
            pl.BlockSpec((None, 1, G), lambda i: (i, 0, 0)),
        ],
        out_specs=pl.BlockSpec((None, N, G), lambda i: (i, 0, 0)),
        out_shape=jax.ShapeDtypeStruct((4, N, G), jnp.float32),
    )(f2, wih, bb)


def _lstm_kernel(t_ref, r_hbm, cnt_ref, whh_ref, bb_ref, h_ref, x_scr, sems,
                 *, base):
    big_t = t_ref[0, 0]
    whh = whh_ref[...]
    bb = bb_ref[...]
    cnts = cnt_ref[...]
    row = lax.broadcasted_iota(jnp.int32, (N, 1), 0)

    def fetch(off, n_t):
        off = pl.multiple_of(off, 8)
        for k in range(NCHUNK):
            @pl.when(k * CW < n_t)
            def _():
                pltpu.make_async_copy(
                    r_hbm.at[pl.ds(base + off + k * CW, CW)],
                    x_scr.at[pl.ds(k * CW, CW)],
                    sems.at[k],
                ).start()

    def drain(off, n_t):
        off = pl.multiple_of(off, 8)
        for k in range(NCHUNK):
            @pl.when(k * CW < n_t)
            def _():
                pltpu.make_async_copy(
                    r_hbm.at[pl.ds(base + off + k * CW, CW)],
                    x_scr.at[pl.ds(k * CW, CW)],
                    sems.at[k],
                ).wait()

    def step(t, carry):
        h, c, off = carry
        n_t = jnp.sum((cnts > t).astype(jnp.int32))
        fetch(off, n_t)
        drain(off, n_t)
        x = x_scr[0:N, :]
        g = (jnp.dot(h, whh, preferred_element_type=jnp.float32)
             + jnp.where(row < n_t, x, bb))
        i = _sigmoid(g[:, 0:H])
        f = _sigmoid(g[:, H:2 * H])
        gg = jnp.tanh(g[:, 2 * H:3 * H])
        o = _sigmoid(g[:, 3 * H:4 * H])
        c = f * c + i * gg
        h = o * jnp.tanh(c)
        npad = ((n_t + 7) // 8) * 8
        return (h, c, off + npad)

    z = jnp.zeros((N, H), jnp.float32)
    h, _, _ = lax.fori_loop(0, big_t, step, (z, z, jnp.int32(0)))
    h_ref[...] = jnp.concatenate([h, jnp.zeros((N, H), jnp.float32)], axis=1)


def _lstm(t_arr, r, cnt2d, whh, bb, base):
    return pl.pallas_call(
        functools.partial(_lstm_kernel, base=base),
        in_specs=[
            pl.BlockSpec(memory_space=pltpu.SMEM),
            pl.BlockSpec(memory_space=pl.ANY),
            pl.BlockSpec(memory_space=pltpu.VMEM),
            pl.BlockSpec(memory_space=pltpu.VMEM),
            pl.BlockSpec(memory_space=pltpu.VMEM),
        ],
        out_shape=jax.ShapeDtypeStruct((N, 2 * H), jnp.float32),
        scratch_shapes=[
            pltpu.VMEM((NCHUNK * CW, G), jnp.float32),
            pltpu.SemaphoreType.DMA((NCHUNK,)),
        ],
    )(t_arr, r, cnt2d, whh, bb)


def _sc_copy(table, gidx2, pos2, out_rows, width, nchunk):
    mesh = plsc.VectorSubcoreMesh(core_axis_name="c", subcore_axis_name="s")

    @functools.partial(
        pl.kernel, mesh=mesh,
        out_type=jax.ShapeDtypeStruct((out_rows, width), jnp.float32),
        scratch_types=[
            pltpu.VMEM((nchunk, SCCH), jnp.int32),
            pltpu.VMEM((nchunk, SCCH), jnp.int32),
            pltpu.VMEM((SCCH, width), jnp.float32),
            pltpu.SemaphoreType.DMA,
        ],
    )
    def k(p_hbm, gidx_hbm, pos_hbm, out_hbm, idx_v, pos_v, rows_v, sem):
        wid = lax.axis_index("s") * 2 + lax.axis_index("c")
        pltpu.sync_copy(gidx_hbm.at[wid], idx_v)
        pltpu.sync_copy(pos_hbm.at[wid], pos_v)
        for i in range(nchunk):
            pltpu.async_copy(p_hbm.at[idx_v.at[i]], rows_v, sem).wait()
            pltpu.async_copy(rows_v, out_hbm.at[pos_v.at[i]], sem).wait()

    return k(table, gidx2, pos2)


def _combine_kernel(h_all, fs, ft, wl, bl, wr, ps_ref, pt_ref,
                    os_ref, ot_ref):
    def lin(k, xd):
        h = h_all[k * UET:k * UET + N, 0:H]
        return (jnp.dot(h, wl[k], preferred_element_type=jnp.float32)
                + bl[k]
                + jnp.dot(xd, wr[k], preferred_element_type=jnp.float32))

    xs = fs[...]
    xt = ft[...]
    o_ss = lin(0, xs)
    o_tt = lin(1, xt)
    o_st = lin(2, xt)
    o_ts = lin(3, xs)
    os_ref[...] = jnp.maximum((o_ss + o_ts) * 0.5 + ps_ref[...], 0.0)
    ot_ref[...] = jnp.maximum((o_tt + o_st) * 0.5 + pt_ref[...], 0.0)


def _combine(h_all, fs, ft, wl, bl, wr, prev_s, prev_t):
    return pl.pallas_call(
        _combine_kernel,
        out_shape=(jax.ShapeDtypeStruct((N, H), jnp.float32),
                   jax.ShapeDtypeStruct((N, H), jnp.float32)),
    )(h_all, fs, ft, wl, bl, wr, prev_s, prev_t)


def _prep_all(edge_indices):
    ar = jnp.arange(N, dtype=jnp.int32)
    dst = jnp.stack([jnp.concatenate([ei[1].astype(jnp.int32), ar])
                     for ei in edge_indices])
    src = jnp.stack([jnp.concatenate([ei[0].astype(jnp.int32), ar])
                     for ei in edge_indices])
    j = jnp.broadcast_to(jnp.arange(NE, dtype=jnp.int32), (4, NE))
    dst_s, src_s = lax.sort((dst, src), dimension=1, is_stable=True,
                            num_keys=1)
    one = jnp.ones((4, 1), jnp.bool_)
    bnd = jnp.concatenate([one, dst_s[:, 1:] != dst_s[:, :-1]], axis=1)
    segstart = lax.cummax(jnp.where(bnd, j, 0), axis=1)
    t = j - segstart
    bnd_next = jnp.concatenate([dst_s[:, 1:] != dst_s[:, :-1], one], axis=1)
    endpos = jnp.flip(
        lax.cummin(jnp.flip(jnp.where(bnd_next, j, NE), axis=1), axis=1),
        axis=1)
    seglen = endpos - segstart + 1
    k2 = (NE - seglen) * 8192 + dst_s
    tp, _k2p, srcp, cntp, dstp = lax.sort((t, k2, src_s, seglen, dst_s),
                                          dimension=1, num_keys=2)
    bndt = jnp.concatenate([one, tp[:, 1:] != tp[:, :-1]], axis=1)
    segstart_t = lax.cummax(jnp.where(bndt, j, 0), axis=1)
    sp_shift = jnp.concatenate(
        [jnp.zeros((4, 1), jnp.int32), segstart_t[:, :-1]], axis=1)
    len_prev = j - sp_shift
    pad_c = jnp.where(bndt & (j > 0), (-len_prev) % 8, 0)
    pos = j + jnp.cumsum(pad_c, axis=1)
    cnt_desc = cntp[:, :N]
    perm = dstp[:, :N]
    big_t = cnt_desc[:, 0:1]
    cnt2d = jnp.pad(cnt_desc, ((0, 0), (0, CPAD - N))).reshape(4, 40, 128)
    koff = jnp.arange(4, dtype=jnp.int32)[:, None]
    gpad = jnp.zeros((4, EPAD - NE), jnp.int32)
    ppad = (165008 + jnp.arange(EPAD - NE, dtype=jnp.int32))[None, :]
    gidx2 = jnp.concatenate([srcp + koff * N, gpad + koff * N], axis=1)
    pos2 = jnp.concatenate([pos + koff * TOTPAD,
                            jnp.broadcast_to(ppad, (4, EPAD - NE))
                            + koff * TOTPAD], axis=1)
    gidx2 = gidx2.reshape(SC_NW, NSCCH, SCCH)
    pos2 = pos2.reshape(SC_NW, NSCCH, SCCH)
    r5 = jnp.broadcast_to(jnp.arange(N, dtype=jnp.int32), (4, N))
    rpad = jnp.broadcast_to(
        (N + jnp.arange(UET - N, dtype=jnp.int32))[None, :], (4, UET - N))
    gidx2u = jnp.concatenate([r5 + koff * N, jnp.zeros((4, UET - N), jnp.int32)
                              + koff * N], axis=1)
    pos2u = jnp.concatenate([perm + koff * UET, rpad + koff * UET], axis=1)
    gidx2u = gidx2u.reshape(SC_NW, NSCCHU, SCCH)
    pos2u = pos2u.reshape(SC_NW, NSCCHU, SCCH)
    return gidx2, pos2, gidx2u, pos2u, cnt2d, big_t


def _layer(prep, fs, ft, lp, prev_s, prev_t):
    gidx2, pos2, gidx2u, pos2u, cnt2d, big_t = prep
    wih = jnp.stack([lp[et]["Wih"] for et in ETS])
    bb = jnp.stack([(lp[et]["bih"] + lp[et]["bhh"]).reshape(1, G) for et in ETS])
    wl = jnp.stack([lp[et]["Wl"] for et in ETS])
    bl = jnp.stack([lp[et]["bl"].reshape(1, H) for et in ETS])
    wr = jnp.stack([lp[et]["Wr"] for et in ETS])
    p_all = _compute_p(fs, ft, wih, bb)
    r_all = _sc_copy(p_all.reshape(4 * N, G), gidx2, pos2,
                     4 * TOTPAD, G, NSCCH)
    hs = []
    for k in range(4):
        t_arr = big_t[k].reshape(1, 1)
        h_perm = _lstm(t_arr, r_all, cnt2d[k], lp[ETS[k]]["Whh"], bb[k],
                       k * TOTPAD)
        hs.append(h_perm)
    h_all = _sc_copy(jnp.concatenate(hs, axis=0), gidx2u, pos2u,
                     4 * UET, 2 * H, NSCCHU)
    return _combine(h_all, fs, ft, wl, bl, wr, prev_s, prev_t)


def kernel(x_source, x_target, edge_index_ss, edge_index_tt, edge_index_st,
           edge_index_ts, edge_attr_ss, edge_attr_tt, edge_attr_st,
           edge_attr_ts, params):
    prep = _prep_all(
        (edge_index_ss, edge_index_tt, edge_index_st, edge_index_ts))
    fs, ft = _encode(x_source, x_target, params["src_enc"], params["tgt_enc"])
    zero = jnp.zeros((N, H), jnp.float32)
    s1, t1 = _layer(prep, fs, ft, params["conv1"], zero, zero)
    s2, t2 = _layer(prep, s1, t1, params["conv2"], s1, t1)
    s3, t3 = _layer(prep, s2, t2, params["conv3"], s2, t2)
    return s3, t3


# speedup vs baseline: 10.9220x; 1.2262x over previous
import functools

import jax
import jax.numpy as jnp
from jax import lax
from jax.experimental import pallas as pl
from jax.experimental.pallas import tpu as pltpu
from jax.experimental.pallas import tpu_sc as plsc

N = 5000
E = 20000
D_IN = 128
H = 64
G = 4 * H
NE = E + N
TOTPAD = 165888
CPAD = 5120
CW = 512
NCHUNK = 10
ETS = ("ss", "tt", "st", "ts")
SC_NW = 32
EPAD = 25600
PERW = 4 * EPAD // SC_NW
SCCH = 128
NSCCH = PERW // SCCH
UET = 5120
NSCCHU = 4 * UET // SC_NW // SCCH


def _sigmoid(x):
    return 1.0 / (1.0 + jnp.exp(-x))


def _enc_kernel(xs_ref, xt_ref, ws1, bs1, ws2, bs2, wt1, bt1, wt2, bt2,
                os_ref, ot_ref):
    hs = jnp.maximum(
        jnp.dot(xs_ref[...], ws1[...], preferred_element_type=jnp.float32)
        + bs1[...], 0.0)
    os_ref[...] = jnp.dot(hs, ws2[...], preferred_element_type=jnp.float32) + bs2[...]
    ht = jnp.maximum(
        jnp.dot(xt_ref[...], wt1[...], preferred_element_type=jnp.float32)
        + bt1[...], 0.0)
    ot_ref[...] = jnp.dot(ht, wt2[...], preferred_element_type=jnp.float32) + bt2[...]


def _encode(x_source, x_target, ps, pt):
    return pl.pallas_call(
        _enc_kernel,
        out_shape=(jax.ShapeDtypeStruct((N, H), jnp.float32),
                   jax.ShapeDtypeStruct((N, H), jnp.float32)),
    )(x_source, x_target,
      ps["W1"], ps["b1"].reshape(1, H), ps["W2"], ps["b2"].reshape(1, H),
      pt["W1"], pt["b1"].reshape(1, H), pt["W2"], pt["b2"].reshape(1, H))


def _p_kernel(f_ref, w_ref, b_ref, p_ref):
    p_ref[...] = (
        jnp.dot(f_ref[...], w_ref[...], preferred_element_type=jnp.float32)
        + b_ref[...])


def _compute_p(fs, ft, wih, bb):
    f2 = jnp.stack([fs, ft])
    return pl.pallas_call(
        _p_kernel,
        grid=(4,),
        in_specs=[
            pl.BlockSpec((None, N, H), lambda i: (i % 2, 0, 0)),
            pl.BlockSpec((None, H, G), lambda i: (i, 0, 0)),
            pl.BlockSpec((None, 1, G), lambda i: (i, 0, 0)),
        ],
        out_specs=pl.BlockSpec((None, N, G), lambda i: (i, 0, 0)),
        out_shape=jax.ShapeDtypeStruct((4, N, G), jnp.float32),
    )(f2, wih, bb)


def _lstm_kernel(t_ref, r_hbm, cnt_ref, whh_ref, bb_ref, h_ref, x_scr, sems,
                 *, base):
    big_t = t_ref[0, 0]
    whh = whh_ref[...]
    bb = bb_ref[...]
    cnts = cnt_ref[...]
    row = lax.broadcasted_iota(jnp.int32, (N, 1), 0)

    def fetch(off, n_t, buf):
        off = pl.multiple_of(off, 8)
        for k in range(NCHUNK):
            @pl.when(k * CW < n_t)
            def _():
                pltpu.make_async_copy(
                    r_hbm.at[pl.ds(base + off + k * CW, CW)],
                    x_scr.at[buf, pl.ds(k * CW, CW)],
                    sems.at[buf, k],
                ).start()

    def drain(off, n_t, buf):
        off = pl.multiple_of(off, 8)
        for k in range(NCHUNK):
            @pl.when(k * CW < n_t)
            def _():
                pltpu.make_async_copy(
                    r_hbm.at[pl.ds(base + off + k * CW, CW)],
                    x_scr.at[buf, pl.ds(k * CW, CW)],
                    sems.at[buf, k],
                ).wait()

    def step(t, carry):
        h, c, off, n_t = carry
        buf = lax.rem(t, 2)
        npad = ((n_t + 7) // 8) * 8
        off1 = off + npad
        n1 = jnp.sum((cnts > t + 1).astype(jnp.int32))
        fetch(off1, n1, 1 - buf)
        drain(off, n_t, buf)
        x = x_scr[buf, 0:N, :]
        g = (jnp.dot(h, whh, preferred_element_type=jnp.float32)
             + jnp.where(row < n_t, x, bb))
        i = _sigmoid(g[:, 0:H])
        f = _sigmoid(g[:, H:2 * H])
        gg = jnp.tanh(g[:, 2 * H:3 * H])
        o = _sigmoid(g[:, 3 * H:4 * H])
        c = f * c + i * gg
        h = o * jnp.tanh(c)
        return (h, c, off1, n1)

    z = jnp.zeros((N, H), jnp.float32)
    n0 = jnp.sum((cnts > 0).astype(jnp.int32))
    fetch(jnp.int32(0), n0, 0)
    h, _, _, _ = lax.fori_loop(0, big_t, step, (z, z, jnp.int32(0), n0))
    h_ref[...] = jnp.concatenate([h, jnp.zeros((N, H), jnp.float32)], axis=1)


def _lstm(t_arr, r, cnt2d, whh, bb, base):
    return pl.pallas_call(
        functools.partial(_lstm_kernel, base=base),
        in_specs=[
            pl.BlockSpec(memory_space=pltpu.SMEM),
            pl.BlockSpec(memory_space=pl.ANY),
            pl.BlockSpec(memory_space=pltpu.VMEM),
            pl.BlockSpec(memory_space=pltpu.VMEM),
            pl.BlockSpec(memory_space=pltpu.VMEM),
        ],
        out_shape=jax.ShapeDtypeStruct((N, 2 * H), jnp.float32),
        scratch_shapes=[
            pltpu.VMEM((2, NCHUNK * CW, G), jnp.float32),
            pltpu.SemaphoreType.DMA((2, NCHUNK)),
        ],
    )(t_arr, r, cnt2d, whh, bb)


def _sc_copy(table, gidx2, pos2, out_rows, width, nchunk):
    mesh = plsc.VectorSubcoreMesh(core_axis_name="c", subcore_axis_name="s")

    @functools.partial(
        pl.kernel, mesh=mesh,
        out_type=jax.ShapeDtypeStruct((out_rows, width), jnp.float32),
        scratch_types=[
            pltpu.VMEM((nchunk, SCCH), jnp.int32),
            pltpu.VMEM((nchunk, SCCH), jnp.int32),
            pltpu.VMEM((SCCH, width), jnp.float32),
            pltpu.SemaphoreType.DMA,
        ],
    )
    def k(p_hbm, gidx_hbm, pos_hbm, out_hbm, idx_v, pos_v, rows_v, sem):
        wid = lax.axis_index("s") * 2 + lax.axis_index("c")
        pltpu.sync_copy(gidx_hbm.at[wid], idx_v)
        pltpu.sync_copy(pos_hbm.at[wid], pos_v)
        for i in range(nchunk):
            pltpu.async_copy(p_hbm.at[idx_v.at[i]], rows_v, sem).wait()
            pltpu.async_copy(rows_v, out_hbm.at[pos_v.at[i]], sem).wait()

    return k(table, gidx2, pos2)


def _combine_kernel(h_all, fs, ft, wl, bl, wr, ps_ref, pt_ref,
                    os_ref, ot_ref):
    def lin(k, xd):
        h = h_all[k * UET:k * UET + N, 0:H]
        return (jnp.dot(h, wl[k], preferred_element_type=jnp.float32)
                + bl[k]
                + jnp.dot(xd, wr[k], preferred_element_type=jnp.float32))

    xs = fs[...]
    xt = ft[...]
    o_ss = lin(0, xs)
    o_tt = lin(1, xt)
    o_st = lin(2, xt)
    o_ts = lin(3, xs)
    os_ref[...] = jnp.maximum((o_ss + o_ts) * 0.5 + ps_ref[...], 0.0)
    ot_ref[...] = jnp.maximum((o_tt + o_st) * 0.5 + pt_ref[...], 0.0)


def _combine(h_all, fs, ft, wl, bl, wr, prev_s, prev_t):
    return pl.pallas_call(
        _combine_kernel,
        out_shape=(jax.ShapeDtypeStruct((N, H), jnp.float32),
                   jax.ShapeDtypeStruct((N, H), jnp.float32)),
    )(h_all, fs, ft, wl, bl, wr, prev_s, prev_t)


def _prep_all(edge_indices):
    ar = jnp.arange(N, dtype=jnp.int32)
    dst = jnp.stack([jnp.concatenate([ei[1].astype(jnp.int32), ar])
                     for ei in edge_indices])
    src = jnp.stack([jnp.concatenate([ei[0].astype(jnp.int32), ar])
                     for ei in edge_indices])
    j = jnp.broadcast_to(jnp.arange(NE, dtype=jnp.int32), (4, NE))
    key1, src_s = lax.sort((dst * 32768 + j, src), dimension=1, num_keys=1)
    dst_s = key1 // 32768
    one = jnp.ones((4, 1), jnp.bool_)
    bnd = jnp.concatenate([one, dst_s[:, 1:] != dst_s[:, :-1]], axis=1)
    segstart = lax.cummax(jnp.where(bnd, j, 0), axis=1)
    t = j - segstart
    bnd_next = jnp.concatenate([dst_s[:, 1:] != dst_s[:, :-1], one], axis=1)
    endpos = jnp.flip(
        lax.cummin(jnp.flip(jnp.where(bnd_next, j, NE), axis=1), axis=1),
        axis=1)
    seglen = endpos - segstart + 1
    k2 = (NE - seglen) * 8192 + dst_s
    tp, k2p, srcp = lax.sort((t, k2, src_s), dimension=1, num_keys=2)
    cntp = NE - k2p // 8192
    dstp = k2p % 8192
    bndt = jnp.concatenate([one, tp[:, 1:] != tp[:, :-1]], axis=1)
    segstart_t = lax.cummax(jnp.where(bndt, j, 0), axis=1)
    sp_shift = jnp.concatenate(
        [jnp.zeros((4, 1), jnp.int32), segstart_t[:, :-1]], axis=1)
    len_prev = j - sp_shift
    pad_c = jnp.where(bndt & (j > 0), (-len_prev) % 8, 0)
    pos = j + jnp.cumsum(pad_c, axis=1)
    cnt_desc = cntp[:, :N]
    perm = dstp[:, :N]
    big_t = cnt_desc[:, 0:1]
    cnt2d = jnp.pad(cnt_desc, ((0, 0), (0, CPAD - N))).reshape(4, 40, 128)
    koff = jnp.arange(4, dtype=jnp.int32)[:, None]
    gpad = jnp.zeros((4, EPAD - NE), jnp.int32)
    ppad = (165008 + jnp.arange(EPAD - NE, dtype=jnp.int32))[None, :]
    gidx2 = jnp.concatenate([srcp + koff * N, gpad + koff * N], axis=1)
    pos2 = jnp.concatenate([pos + koff * TOTPAD,
                            jnp.broadcast_to(ppad, (4, EPAD - NE))
                            + koff * TOTPAD], axis=1)
    gidx2 = gidx2.reshape(SC_NW, NSCCH, SCCH)
    pos2 = pos2.reshape(SC_NW, NSCCH, SCCH)
    r5 = jnp.broadcast_to(jnp.arange(N, dtype=jnp.int32), (4, N))
    rpad = jnp.broadcast_to(
        (N + jnp.arange(UET - N, dtype=jnp.int32))[None, :], (4, UET - N))
    gidx2u = jnp.concatenate([r5 + koff * N, jnp.zeros((4, UET - N), jnp.int32)
                              + koff * N], axis=1)
    pos2u = jnp.concatenate([perm + koff * UET, rpad + koff * UET], axis=1)
    gidx2u = gidx2u.reshape(SC_NW, NSCCHU, SCCH)
    pos2u = pos2u.reshape(SC_NW, NSCCHU, SCCH)
    return gidx2, pos2, gidx2u, pos2u, cnt2d, big_t


def _layer(prep, fs, ft, lp, prev_s, prev_t):
    gidx2, pos2, gidx2u, pos2u, cnt2d, big_t = prep
    wih = jnp.stack([lp[et]["Wih"] for et in ETS])
    bb = jnp.stack([(lp[et]["bih"] + lp[et]["bhh"]).reshape(1, G) for et in ETS])
    wl = jnp.stack([lp[et]["Wl"] for et in ETS])
    bl = jnp.stack([lp[et]["bl"].reshape(1, H) for et in ETS])
    wr = jnp.stack([lp[et]["Wr"] for et in ETS])
    p_all = _compute_p(fs, ft, wih, bb)
    r_all = _sc_copy(p_all.reshape(4 * N, G), gidx2, pos2,
                     4 * TOTPAD, G, NSCCH)
    hs = []
    for k in range(4):
        t_arr = big_t[k].reshape(1, 1)
        h_perm = _lstm(t_arr, r_all, cnt2d[k], lp[ETS[k]]["Whh"], bb[k],
                       k * TOTPAD)
        hs.append(h_perm)
    h_all = _sc_copy(jnp.concatenate(hs, axis=0), gidx2u, pos2u,
                     4 * UET, 2 * H, NSCCHU)
    return _combine(h_all, fs, ft, wl, bl, wr, prev_s, prev_t)


def kernel(x_source, x_target, edge_index_ss, edge_index_tt, edge_index_st,
           edge_index_ts, edge_attr_ss, edge_attr_tt, edge_attr_st,
           edge_attr_ts, params):
    prep = _prep_all(
        (edge_index_ss, edge_index_tt, edge_index_st, edge_index_ts))
    fs, ft = _encode(x_source, x_target, params["src_enc"], params["tgt_enc"])
    zero = jnp.zeros((N, H), jnp.float32)
    s1, t1 = _layer(prep, fs, ft, params["conv1"], zero, zero)
    s2, t2 = _layer(prep, s1, t1, params["conv2"], s1, t1)
    s3, t3 = _layer(prep, s2, t2, params["conv3"], s2, t2)
    return s3, t3


# speedup vs baseline: 13.1593x; 1.2049x over previous
import functools

import jax
import jax.numpy as jnp
from jax import lax
from jax.experimental import pallas as pl
from jax.experimental.pallas import tpu as pltpu
from jax.experimental.pallas import tpu_sc as plsc

N = 5000
E = 20000
D_IN = 128
H = 64
G = 4 * H
NE = E + N
TOTPAD = 165888
CPAD = 5120
CW = 512
NCHUNK = 10
ETS = ("ss", "tt", "st", "ts")
SC_NW = 32
EPAD = 25088
SCCH = 112
NSCCH = EPAD // SC_NW // SCCH
UET = 5120
UCH = 128
NSCCHU = 4 * UET // SC_NW // UCH


def _sigmoid(x):
    return 1.0 / (1.0 + jnp.exp(-x))


def _enc_kernel(xs_ref, xt_ref, ws1, bs1, ws2, bs2, wt1, bt1, wt2, bt2,
                os_ref, ot_ref):
    hs = jnp.maximum(
        jnp.dot(xs_ref[...], ws1[...], preferred_element_type=jnp.float32)
        + bs1[...], 0.0)
    os_ref[...] = jnp.dot(hs, ws2[...], preferred_element_type=jnp.float32) + bs2[...]
    ht = jnp.maximum(
        jnp.dot(xt_ref[...], wt1[...], preferred_element_type=jnp.float32)
        + bt1[...], 0.0)
    ot_ref[...] = jnp.dot(ht, wt2[...], preferred_element_type=jnp.float32) + bt2[...]


def _encode(x_source, x_target, ps, pt):
    return pl.pallas_call(
        _enc_kernel,
        out_shape=(jax.ShapeDtypeStruct((N, H), jnp.float32),
                   jax.ShapeDtypeStruct((N, H), jnp.float32)),
    )(x_source, x_target,
      ps["W1"], ps["b1"].reshape(1, H), ps["W2"], ps["b2"].reshape(1, H),
      pt["W1"], pt["b1"].reshape(1, H), pt["W2"], pt["b2"].reshape(1, H))


def _p_kernel(f_ref, w_ref, b_ref, p_ref):
    p_ref[...] = (
        jnp.dot(f_ref[...], w_ref[...], preferred_element_type=jnp.float32)
        + b_ref[...])


def _compute_p(fs, ft, wih, bb):
    f2 = jnp.stack([fs, ft])
    return pl.pallas_call(
        _p_kernel,
        grid=(4,),
        in_specs=[
            pl.BlockSpec((None, N, H), lambda i: (i % 2, 0, 0)),
            pl.BlockSpec((None, H, G), lambda i: (i, 0, 0)),
            pl.BlockSpec((None, 1, G), lambda i: (i, 0, 0)),
        ],
        out_specs=pl.BlockSpec((None, N, G), lambda i: (i, 0, 0)),
        out_shape=jax.ShapeDtypeStruct((4, N, G), jnp.float32),
    )(f2, wih, bb)


def _lstm_kernel(t_ref, r_hbm, cnt_ref, whh_ref, bb_ref, h_ref, x_scr, sems,
                 *, base):
    big_t = t_ref[0, 0]
    whh = whh_ref[...]
    bb = bb_ref[...]
    cnts = cnt_ref[...]
    row = lax.broadcasted_iota(jnp.int32, (N, 1), 0)

    def fetch(off, n_t, buf):
        off = pl.multiple_of(off, 8)
        for k in range(NCHUNK):
            @pl.when(k * CW < n_t)
            def _():
                pltpu.make_async_copy(
                    r_hbm.at[pl.ds(base + off + k * CW, CW)],
                    x_scr.at[buf, pl.ds(k * CW, CW)],
                    sems.at[buf, k],
                ).start()

    def drain(off, n_t, buf):
        off = pl.multiple_of(off, 8)
        for k in range(NCHUNK):
            @pl.when(k * CW < n_t)
            def _():
                pltpu.make_async_copy(
                    r_hbm.at[pl.ds(base + off + k * CW, CW)],
                    x_scr.at[buf, pl.ds(k * CW, CW)],
                    sems.at[buf, k],
                ).wait()

    def step(t, carry):
        h, c, off, n_t = carry
        buf = lax.rem(t, 2)
        npad = ((n_t + 7) // 8) * 8
        off1 = off + npad
        n1 = jnp.sum((cnts > t + 1).astype(jnp.int32))
        fetch(off1, n1, 1 - buf)
        drain(off, n_t, buf)
        x = x_scr[buf, 0:N, :]
        g = (jnp.dot(h, whh, preferred_element_type=jnp.float32)
             + jnp.where(row < n_t, x, bb))
        i = _sigmoid(g[:, 0:H])
        f = _sigmoid(g[:, H:2 * H])
        gg = jnp.tanh(g[:, 2 * H:3 * H])
        o = _sigmoid(g[:, 3 * H:4 * H])
        c = f * c + i * gg
        h = o * jnp.tanh(c)
        return (h, c, off1, n1)

    z = jnp.zeros((N, H), jnp.float32)
    n0 = jnp.sum((cnts > 0).astype(jnp.int32))
    fetch(jnp.int32(0), n0, 0)
    h, _, _, _ = lax.fori_loop(0, big_t, step, (z, z, jnp.int32(0), n0))
    h_ref[...] = jnp.concatenate([h, jnp.zeros((N, H), jnp.float32)], axis=1)


def _lstm(t_arr, r, cnt2d, whh, bb, base):
    return pl.pallas_call(
        functools.partial(_lstm_kernel, base=base),
        in_specs=[
            pl.BlockSpec(memory_space=pltpu.SMEM),
            pl.BlockSpec(memory_space=pl.ANY),
            pl.BlockSpec(memory_space=pltpu.VMEM),
            pl.BlockSpec(memory_space=pltpu.VMEM),
            pl.BlockSpec(memory_space=pltpu.VMEM),
        ],
        out_shape=jax.ShapeDtypeStruct((N, 2 * H), jnp.float32),
        scratch_shapes=[
            pltpu.VMEM((2, NCHUNK * CW, G), jnp.float32),
            pltpu.SemaphoreType.DMA((2, NCHUNK)),
        ],
    )(t_arr, r, cnt2d, whh, bb)


def _sc_copy(table, gidx2, pos2, out_rows, width):
    _, nchunk, scch = gidx2.shape
    mesh = plsc.VectorSubcoreMesh(core_axis_name="c", subcore_axis_name="s")

    @functools.partial(
        pl.kernel, mesh=mesh,
        out_type=jax.ShapeDtypeStruct((out_rows, width), jnp.float32),
        scratch_types=[
            pltpu.VMEM((nchunk, scch), jnp.int32),
            pltpu.VMEM((nchunk, scch), jnp.int32),
            pltpu.VMEM((scch, width), jnp.float32),
            pltpu.SemaphoreType.DMA,
        ],
    )
    def k(p_hbm, gidx_hbm, pos_hbm, out_hbm, idx_v, pos_v, rows_v, sem):
        wid = lax.axis_index("s") * 2 + lax.axis_index("c")
        pltpu.sync_copy(gidx_hbm.at[wid], idx_v)
        pltpu.sync_copy(pos_hbm.at[wid], pos_v)
        for i in range(nchunk):
            pltpu.async_copy(p_hbm.at[idx_v.at[i]], rows_v, sem).wait()
            pltpu.async_copy(rows_v, out_hbm.at[pos_v.at[i]], sem).wait()

    return k(table, gidx2, pos2)


def _combine_kernel(h_all, fs, ft, wl, bl, wr, ps_ref, pt_ref,
                    os_ref, ot_ref):
    def lin(k, xd):
        h = h_all[k * UET:k * UET + N, 0:H]
        return (jnp.dot(h, wl[k], preferred_element_type=jnp.float32)
                + bl[k]
                + jnp.dot(xd, wr[k], preferred_element_type=jnp.float32))

    xs = fs[...]
    xt = ft[...]
    o_ss = lin(0, xs)
    o_tt = lin(1, xt)
    o_st = lin(2, xt)
    o_ts = lin(3, xs)
    os_ref[...] = jnp.maximum((o_ss + o_ts) * 0.5 + ps_ref[...], 0.0)
    ot_ref[...] = jnp.maximum((o_tt + o_st) * 0.5 + pt_ref[...], 0.0)


def _combine(h_all, fs, ft, wl, bl, wr, prev_s, prev_t):
    return pl.pallas_call(
        _combine_kernel,
        out_shape=(jax.ShapeDtypeStruct((N, H), jnp.float32),
                   jax.ShapeDtypeStruct((N, H), jnp.float32)),
    )(h_all, fs, ft, wl, bl, wr, prev_s, prev_t)


def _prep_all(edge_indices):
    ar = jnp.arange(N, dtype=jnp.int32)
    dst = jnp.stack([jnp.concatenate([ei[1].astype(jnp.int32), ar])
                     for ei in edge_indices])
    src = jnp.stack([jnp.concatenate([ei[0].astype(jnp.int32), ar])
                     for ei in edge_indices])
    j = jnp.broadcast_to(jnp.arange(NE, dtype=jnp.int32), (4, NE))
    key1, src_s = lax.sort((dst * 32768 + j, src), dimension=1, num_keys=1)
    dst_s = key1 // 32768
    one = jnp.ones((4, 1), jnp.bool_)
    bnd = jnp.concatenate([one, dst_s[:, 1:] != dst_s[:, :-1]], axis=1)
    segstart = lax.cummax(jnp.where(bnd, j, 0), axis=1)
    t = j - segstart
    bnd_next = jnp.concatenate([dst_s[:, 1:] != dst_s[:, :-1], one], axis=1)
    endpos = jnp.flip(
        lax.cummin(jnp.flip(jnp.where(bnd_next, j, NE), axis=1), axis=1),
        axis=1)
    seglen = endpos - segstart + 1
    k2 = (NE - seglen) * 8192 + dst_s
    tp, k2p, srcp = lax.sort((t, k2, src_s), dimension=1, num_keys=2)
    cntp = NE - k2p // 8192
    dstp = k2p % 8192
    bndt = jnp.concatenate([one, tp[:, 1:] != tp[:, :-1]], axis=1)
    segstart_t = lax.cummax(jnp.where(bndt, j, 0), axis=1)
    sp_shift = jnp.concatenate(
        [jnp.zeros((4, 1), jnp.int32), segstart_t[:, :-1]], axis=1)
    len_prev = j - sp_shift
    pad_c = jnp.where(bndt & (j > 0), (-len_prev) % 8, 0)
    pos = j + jnp.cumsum(pad_c, axis=1)
    cnt_desc = cntp[:, :N]
    perm = dstp[:, :N]
    big_t = cnt_desc[:, 0:1]
    cnt2d = jnp.pad(cnt_desc, ((0, 0), (0, CPAD - N))).reshape(4, 40, 128)
    koff = jnp.arange(4, dtype=jnp.int32)[:, None]
    gpad = jnp.zeros((4, EPAD - NE), jnp.int32)
    ppad = (165008 + jnp.arange(EPAD - NE, dtype=jnp.int32))[None, :]
    gidx2 = jnp.concatenate([srcp + koff * N, gpad + koff * N], axis=1)
    pos2 = jnp.concatenate([pos,
                            jnp.broadcast_to(ppad, (4, EPAD - NE))], axis=1)
    gidx2 = gidx2.reshape(4, SC_NW, NSCCH, SCCH)
    pos2 = pos2.reshape(4, SC_NW, NSCCH, SCCH)
    r5 = jnp.broadcast_to(jnp.arange(N, dtype=jnp.int32), (4, N))
    rpad = jnp.broadcast_to(
        (N + jnp.arange(UET - N, dtype=jnp.int32))[None, :], (4, UET - N))
    gidx2u = jnp.concatenate([r5 + koff * N, jnp.zeros((4, UET - N), jnp.int32)
                              + koff * N], axis=1)
    pos2u = jnp.concatenate([perm + koff * UET, rpad + koff * UET], axis=1)
    gidx2u = gidx2u.reshape(SC_NW, NSCCHU, UCH)
    pos2u = pos2u.reshape(SC_NW, NSCCHU, UCH)
    return gidx2, pos2, gidx2u, pos2u, cnt2d, big_t


def _layer(prep, fs, ft, lp, prev_s, prev_t):
    gidx2, pos2, gidx2u, pos2u, cnt2d, big_t = prep
    wih = jnp.stack([lp[et]["Wih"] for et in ETS])
    bb = jnp.stack([(lp[et]["bih"] + lp[et]["bhh"]).reshape(1, G) for et in ETS])
    wl = jnp.stack([lp[et]["Wl"] for et in ETS])
    bl = jnp.stack([lp[et]["bl"].reshape(1, H) for et in ETS])
    wr = jnp.stack([lp[et]["Wr"] for et in ETS])
    p_all = _compute_p(fs, ft, wih, bb)
    p_flat = p_all.reshape(4 * N, G)
    rs = [_sc_copy(p_flat, gidx2[k], pos2[k], TOTPAD, G) for k in range(4)]
    hs = []
    for k in range(4):
        t_arr = big_t[k].reshape(1, 1)
        h_perm = _lstm(t_arr, rs[k], cnt2d[k], lp[ETS[k]]["Whh"], bb[k], 0)
        hs.append(h_perm)
    h_all = _sc_copy(jnp.concatenate(hs, axis=0), gidx2u, pos2u,
                     4 * UET, 2 * H)
    return _combine(h_all, fs, ft, wl, bl, wr, prev_s, prev_t)


def kernel(x_source, x_target, edge_index_ss, edge_index_tt, edge_index_st,
           edge_index_ts, edge_attr_ss, edge_attr_tt, edge_attr_st,
           edge_attr_ts, params):
    prep = _prep_all(
        (edge_index_ss, edge_index_tt, edge_index_st, edge_index_ts))
    fs, ft = _encode(x_source, x_target, params["src_enc"], params["tgt_enc"])
    zero = jnp.zeros((N, H), jnp.float32)
    s1, t1 = _layer(prep, fs, ft, params["conv1"], zero, zero)
    s2, t2 = _layer(prep, s1, t1, params["conv2"], s1, t1)
    s3, t3 = _layer(prep, s2, t2, params["conv3"], s2, t2)
    return s3, t3
